# baseline (speedup 1.0000x reference)
import functools

import jax
import jax.numpy as jnp
from jax import lax
from jax.experimental import pallas as pl
from jax.experimental.pallas import tpu as pltpu
from jax.experimental.pallas import tpu_sc as plsc

D = 1024
DFF = 4 * D
E = 8
N = 2048
K = 2
A = N * K
BT = 256
NB = A // BT + E
S = NB * BT
NW = 32
APW = A // NW
CH = 64
TPW = N // NW
CH2 = 16


def _erf(z):
    s = jnp.sign(z)
    a = jnp.abs(z)
    t = 1.0 / (1.0 + 0.3275911 * a)
    poly = t * (0.254829592 + t * (-0.284496736 + t * (
        1.421413741 + t * (-1.453152027 + t * 1.061405429))))
    return s * (1.0 - poly * jnp.exp(-a * a))


def _gelu_exact(h):
    return 0.5 * h * (1.0 + _erf(h * 0.7071067811865476))


def _gelu_tanh(h):
    u = 0.7978845608028654 * (h + 0.044715 * (h * h * h))
    return 0.5 * h * (1.0 + jnp.tanh(u))


def _cumsum0(m):
    c = m
    s = 1
    while s < m.shape[0]:
        c = c + jnp.concatenate(
            [jnp.zeros((s, m.shape[1]), m.dtype), c[:-s]], axis=0)
        s *= 2
    return c


def _cumsum1(m):
    c = m
    s = 1
    while s < m.shape[1]:
        c = c + jnp.concatenate(
            [jnp.zeros((m.shape[0], s), m.dtype), c[:, :-s]], axis=1)
        s *= 2
    return c


def _route_body(x_ref, wg_ref, bg_ref, dst_ref, wts_ref, bexp_ref, bact_ref):
    x = x_ref[...]
    gates = lax.dot_general(
        x, wg_ref[...], (((1,), (0,)), ((), ())),
        preferred_element_type=jnp.float32) + bg_ref[...]
    lane = lax.broadcasted_iota(jnp.int32, (N, E), 1)
    big = jnp.int32(1_000_000)
    m1 = jnp.max(gates, axis=1, keepdims=True)
    a1 = jnp.min(jnp.where(gates == m1, lane, big), axis=1, keepdims=True)
    gates2 = jnp.where(lane == a1, -jnp.inf, gates)
    m2 = jnp.max(gates2, axis=1, keepdims=True)
    a2 = jnp.min(jnp.where(gates2 == m2, lane, big), axis=1, keepdims=True)
    w1 = 1.0 / (1.0 + jnp.exp(m2 - m1))
    w2 = 1.0 - w1

    oh0 = (lane == a1).astype(jnp.float32)
    oh1 = (lane == a2).astype(jnp.float32)
    cum0 = _cumsum0(oh0)
    cum1 = _cumsum0(oh1)
    cnt0 = cum0[-1:, :]
    cnt1 = cum1[-1:, :]
    cnt = (cnt0 + cnt1).astype(jnp.int32)
    cntpad = ((cnt + (BT - 1)) // BT) * BT
    cum_end = _cumsum1(cntpad)
    offpad = (cum_end - cntpad).astype(jnp.float32)
    cnt0f = cnt0

    off0 = jnp.sum(oh0 * offpad, axis=1, keepdims=True)
    off1 = jnp.sum(oh1 * offpad, axis=1, keepdims=True)
    rank0 = jnp.sum(oh0 * cum0, axis=1, keepdims=True) - 1.0
    rank1 = jnp.sum(oh1 * cum1, axis=1, keepdims=True) - 1.0
    base1 = jnp.sum(oh1 * cnt0f, axis=1, keepdims=True)
    pos0 = off0 + rank0
    pos1 = off1 + base1 + rank1
    dst_ref[...] = jnp.concatenate([pos0, pos1], axis=1).astype(jnp.int32)
    wts_ref[...] = jnp.concatenate([w1, w2], axis=1)

    biota = lax.broadcasted_iota(jnp.int32, (1, NB), 1) * BT
    acc = jnp.zeros((1, NB), jnp.int32)
    for e in range(E):
        ce = lax.slice(cum_end, (0, e), (1, e + 1))
        acc = acc + (biota >= ce).astype(jnp.int32)
    total = lax.slice(cum_end, (0, E - 1), (1, E))
    bexp_ref[...] = jnp.minimum(acc, E - 1)
    bact_ref[...] = (biota < total).astype(jnp.int32)


def _ffn_body(bexp_ref, bact_ref, xg_ref, w1_ref, b1_ref, w2_ref, b2_ref,
              ws_ref, out_ref):
    b = pl.program_id(0)

    @pl.when(bact_ref[b] != 0)
    def _():
        xb = xg_ref[...]
        h = lax.dot_general(
            xb, w1_ref[0], (((1,), (0,)), ((), ())),
            preferred_element_type=jnp.float32) + b1_ref[0]
        h = _gelu_tanh(h)
        y = lax.dot_general(
            h.astype(jnp.bfloat16), w2_ref[0], (((1,), (0,)), ((), ())),
            preferred_element_type=jnp.float32) + b2_ref[0]
        out_ref[...] = y * ws_ref[:, :1]


@functools.cache
def _sc_kernels():
    mesh = plsc.VectorSubcoreMesh(core_axis_name="c", subcore_axis_name="s")

    @functools.partial(
        pl.kernel, mesh=mesh,
        out_type=[jax.ShapeDtypeStruct((S, D // 2), jnp.int32),
                  jax.ShapeDtypeStruct((S, 128), jnp.float32)],
        scratch_types=[pltpu.VMEM((CH,), jnp.int32),
                       pltpu.VMEM((CH,), jnp.int32),
                       pltpu.VMEM((CH, D // 2), jnp.int32),
                       pltpu.VMEM((CH, D // 2), jnp.int32),
                       pltpu.VMEM((APW,), jnp.int32),
                       pltpu.VMEM((APW, 128), jnp.float32),
                       pltpu.SemaphoreType.DMA,
                       pltpu.SemaphoreType.DMA,
                       pltpu.SemaphoreType.DMA],
    )
    def _dispatch(x_hbm, dst3_hbm, dstw_hbm, wf3_hbm, xg_hbm, ws_hbm,
                  idx_a, idx_b, rows_a, rows_b, idxw_v, wrows_v,
                  sem_a, sem_b, sem_w):
        wid = lax.axis_index("s") * 2 + lax.axis_index("c")
        n0 = (wid % (N // APW)) * APW
        pltpu.sync_copy(dst3_hbm.at[wid, 0], idx_a)
        pltpu.sync_copy(x_hbm.at[pl.ds(n0, CH)], rows_a)
        cp_a = pltpu.async_copy(rows_a, xg_hbm.at[idx_a], sem_a)
        pltpu.sync_copy(dst3_hbm.at[wid, 1], idx_b)
        pltpu.sync_copy(x_hbm.at[pl.ds(n0 + CH, CH)], rows_b)
        cp_b = pltpu.async_copy(rows_b, xg_hbm.at[idx_b], sem_b)
        pltpu.sync_copy(dstw_hbm.at[wid], idxw_v)
        pltpu.sync_copy(wf3_hbm.at[wid], wrows_v)
        cp_w = pltpu.async_copy(wrows_v, ws_hbm.at[idxw_v], sem_w)
        cp_a.wait()
        cp_b.wait()
        cp_w.wait()

    nch = TPW // CH2

    @functools.partial(
        pl.kernel, mesh=mesh,
        out_type=jax.ShapeDtypeStruct((N, D), jnp.float32),
        scratch_types=[pltpu.VMEM((2, CH2), jnp.int32),
                       pltpu.VMEM((2, CH2), jnp.int32),
                       pltpu.VMEM((2, CH2, D), jnp.float32),
                       pltpu.VMEM((2, CH2, D), jnp.float32),
                       pltpu.SemaphoreType.DMA,
                       pltpu.SemaphoreType.DMA],
    )
    def _combine(yg_hbm, dstT_hbm, out_hbm, i0_v, i1_v, r0_v, r1_v,
                 sem_a, sem_b):
        wid = lax.axis_index("s") * 2 + lax.axis_index("c")

        def start(c, buf):
            t0 = wid * TPW + c * CH2
            pltpu.sync_copy(dstT_hbm.at[0, pl.ds(t0, CH2)], i0_v.at[buf])
            pltpu.sync_copy(dstT_hbm.at[1, pl.ds(t0, CH2)], i1_v.at[buf])
            ca = pltpu.async_copy(yg_hbm.at[i0_v.at[buf]], r0_v.at[buf], sem_a)
            cb = pltpu.async_copy(yg_hbm.at[i1_v.at[buf]], r1_v.at[buf], sem_b)
            return ca, cb

        def finish(c, buf, cps):
            cps[0].wait()
            cps[1].wait()

            def rbody(r, carry):
                for j in range(D // 16):
                    sl = pl.ds(j * 16, 16)
                    r0_v[buf, r, sl] = r0_v[buf, r, sl] + r1_v[buf, r, sl]
                return carry

            lax.fori_loop(0, CH2, rbody, 0)
            t0 = wid * TPW + c * CH2
            pltpu.sync_copy(r0_v.at[buf], out_hbm.at[pl.ds(t0, CH2)])

        cps = start(0, 0)
        for c in range(nch):
            nxt = start(c + 1, (c + 1) % 2) if c + 1 < nch else None
            finish(c, c % 2, cps)
            cps = nxt

    return _dispatch, _combine


def kernel(x, Wg, bg, W1, b1, W2, b2):
    orig_shape = x.shape
    xf = x.reshape(-1, orig_shape[-1])
    bg2 = bg.reshape(1, E)
    w1b = W1.astype(jnp.bfloat16)
    w2b = W2.astype(jnp.bfloat16)
    b1r = b1.reshape(E, 1, DFF)
    b2r = b2.reshape(E, 1, D)

    dst, wts, bexp, bact = pl.pallas_call(
        _route_body,
        in_specs=[
            pl.BlockSpec((N, D), lambda: (0, 0)),
            pl.BlockSpec((D, E), lambda: (0, 0)),
            pl.BlockSpec((1, E), lambda: (0, 0)),
        ],
        out_specs=[
            pl.BlockSpec((N, K), lambda: (0, 0)),
            pl.BlockSpec((N, K), lambda: (0, 0)),
            pl.BlockSpec((1, NB), lambda: (0, 0)),
            pl.BlockSpec((1, NB), lambda: (0, 0)),
        ],
        out_shape=[
            jax.ShapeDtypeStruct((N, K), jnp.int32),
            jax.ShapeDtypeStruct((N, K), jnp.float32),
            jax.ShapeDtypeStruct((1, NB), jnp.int32),
            jax.ShapeDtypeStruct((1, NB), jnp.int32),
        ],
    )(xf, Wg, bg2)

    dstT = dst.T
    dst3 = dstT.reshape(NW, APW // CH, CH)
    dstw = dstT.reshape(NW, APW)
    wf3 = jnp.broadcast_to(wts.T.reshape(NW, APW, 1), (NW, APW, 128))

    _dispatch, _combine = _sc_kernels()
    xbf = xf.astype(jnp.bfloat16)
    xi = lax.bitcast_convert_type(xbf.reshape(N, D // 2, 2), jnp.int32)
    xgi, wsort = _dispatch(xi, dst3, dstw, wf3)
    xg = lax.bitcast_convert_type(xgi, jnp.bfloat16).reshape(S, D)

    grid_spec = pltpu.PrefetchScalarGridSpec(
        num_scalar_prefetch=2,
        grid=(NB,),
        in_specs=[
            pl.BlockSpec((BT, D), lambda b, be, ba: (b, 0)),
            pl.BlockSpec((1, D, DFF), lambda b, be, ba: (be[b], 0, 0)),
            pl.BlockSpec((1, 1, DFF), lambda b, be, ba: (be[b], 0, 0)),
            pl.BlockSpec((1, DFF, D), lambda b, be, ba: (be[b], 0, 0)),
            pl.BlockSpec((1, 1, D), lambda b, be, ba: (be[b], 0, 0)),
            pl.BlockSpec((BT, 128), lambda b, be, ba: (b, 0)),
        ],
        out_specs=pl.BlockSpec((BT, D), lambda b, be, ba: (b, 0)),
    )
    yg = pl.pallas_call(
        _ffn_body,
        grid_spec=grid_spec,
        out_shape=jax.ShapeDtypeStruct((S, D), jnp.float32),
        compiler_params=pltpu.CompilerParams(
            dimension_semantics=("arbitrary",)),
    )(bexp.reshape(NB), bact.reshape(NB), xg, w1b, b1r, w2b, b2r, wsort)

    out = _combine(yg, dstT)
    return out.reshape(orig_shape)


# speedup vs baseline: 1.5700x; 1.5700x over previous
import functools

import jax
import jax.numpy as jnp
from jax import lax
from jax.experimental import pallas as pl
from jax.experimental.pallas import tpu as pltpu
from jax.experimental.pallas import tpu_sc as plsc

D = 1024
DFF = 4 * D
E = 8
N = 2048
K = 2
A = N * K
BT = 256
NB = A // BT + E
S = NB * BT
NW = 32
APW = A // NW
CH = 32
TPW = N // NW
CH2 = 16


def _erf(z):
    s = jnp.sign(z)
    a = jnp.abs(z)
    t = 1.0 / (1.0 + 0.3275911 * a)
    poly = t * (0.254829592 + t * (-0.284496736 + t * (
        1.421413741 + t * (-1.453152027 + t * 1.061405429))))
    return s * (1.0 - poly * jnp.exp(-a * a))


def _gelu_exact(h):
    return 0.5 * h * (1.0 + _erf(h * 0.7071067811865476))


def _gelu_tanh(h):
    u = 0.7978845608028654 * (h + 0.044715 * (h * h * h))
    return 0.5 * h * (1.0 + jnp.tanh(u))


def _cumsum0(m):
    c = m
    s = 1
    while s < m.shape[0]:
        c = c + jnp.concatenate(
            [jnp.zeros((s, m.shape[1]), m.dtype), c[:-s]], axis=0)
        s *= 2
    return c


def _cumsum1(m):
    c = m
    s = 1
    while s < m.shape[1]:
        c = c + jnp.concatenate(
            [jnp.zeros((m.shape[0], s), m.dtype), c[:, :-s]], axis=1)
        s *= 2
    return c


def _route_body(x_ref, wg_ref, bg_ref, dst_ref, wts_ref, bexp_ref, bact_ref):
    x = x_ref[...]
    gates = lax.dot_general(
        x, wg_ref[...], (((1,), (0,)), ((), ())),
        preferred_element_type=jnp.float32) + bg_ref[...]
    lane = lax.broadcasted_iota(jnp.int32, (N, E), 1)
    big = jnp.int32(1_000_000)
    m1 = jnp.max(gates, axis=1, keepdims=True)
    a1 = jnp.min(jnp.where(gates == m1, lane, big), axis=1, keepdims=True)
    gates2 = jnp.where(lane == a1, -jnp.inf, gates)
    m2 = jnp.max(gates2, axis=1, keepdims=True)
    a2 = jnp.min(jnp.where(gates2 == m2, lane, big), axis=1, keepdims=True)
    w1 = 1.0 / (1.0 + jnp.exp(m2 - m1))
    w2 = 1.0 - w1

    oh0 = (lane == a1).astype(jnp.float32)
    oh1 = (lane == a2).astype(jnp.float32)
    cum0 = _cumsum0(oh0)
    cum1 = _cumsum0(oh1)
    cnt0 = cum0[-1:, :]
    cnt1 = cum1[-1:, :]
    cnt = (cnt0 + cnt1).astype(jnp.int32)
    cntpad = ((cnt + (BT - 1)) // BT) * BT
    cum_end = _cumsum1(cntpad)
    offpad = (cum_end - cntpad).astype(jnp.float32)
    cnt0f = cnt0

    off0 = jnp.sum(oh0 * offpad, axis=1, keepdims=True)
    off1 = jnp.sum(oh1 * offpad, axis=1, keepdims=True)
    rank0 = jnp.sum(oh0 * cum0, axis=1, keepdims=True) - 1.0
    rank1 = jnp.sum(oh1 * cum1, axis=1, keepdims=True) - 1.0
    base1 = jnp.sum(oh1 * cnt0f, axis=1, keepdims=True)
    pos0 = off0 + rank0
    pos1 = off1 + base1 + rank1
    dst_ref[...] = jnp.concatenate([pos0, pos1], axis=1).astype(jnp.int32)
    wts_ref[...] = jnp.concatenate([w1, w2], axis=1)

    biota = lax.broadcasted_iota(jnp.int32, (1, NB), 1) * BT
    acc = jnp.zeros((1, NB), jnp.int32)
    for e in range(E):
        ce = lax.slice(cum_end, (0, e), (1, e + 1))
        acc = acc + (biota >= ce).astype(jnp.int32)
    total = lax.slice(cum_end, (0, E - 1), (1, E))
    bexp_ref[...] = jnp.minimum(acc, E - 1)
    bact_ref[...] = (biota < total).astype(jnp.int32)


def _ffn_body(bexp_ref, bact_ref, xg_ref, w1_ref, b1_ref, w2_ref, b2_ref,
              ws_ref, out_ref):
    b = pl.program_id(0)

    @pl.when(bact_ref[b] != 0)
    def _():
        xb = xg_ref[...].astype(jnp.bfloat16)
        h = lax.dot_general(
            xb, w1_ref[0], (((1,), (0,)), ((), ())),
            preferred_element_type=jnp.float32) + b1_ref[0]
        h = _gelu_tanh(h)
        y = lax.dot_general(
            h.astype(jnp.bfloat16), w2_ref[0], (((1,), (0,)), ((), ())),
            preferred_element_type=jnp.float32) + b2_ref[0]
        out_ref[...] = y * ws_ref[:, :1]


@functools.cache
def _sc_kernels():
    mesh = plsc.VectorSubcoreMesh(core_axis_name="c", subcore_axis_name="s")

    @functools.partial(
        pl.kernel, mesh=mesh,
        out_type=[jax.ShapeDtypeStruct((S, D), jnp.float32),
                  jax.ShapeDtypeStruct((S, 128), jnp.float32)],
        scratch_types=[pltpu.VMEM((2, CH), jnp.int32),
                       pltpu.VMEM((2, CH, D), jnp.float32),
                       pltpu.VMEM((APW,), jnp.int32),
                       pltpu.VMEM((APW, 128), jnp.float32),
                       pltpu.SemaphoreType.DMA,
                       pltpu.SemaphoreType.DMA,
                       pltpu.SemaphoreType.DMA],
    )
    def _dispatch(x_hbm, dst3_hbm, dstw_hbm, wf3_hbm, xg_hbm, ws_hbm,
                  idx_v, rows_v, idxw_v, wrows_v, sem_a, sem_b, sem_w):
        wid = lax.axis_index("s") * 2 + lax.axis_index("c")
        n0 = (wid % (N // APW)) * APW
        nchd = APW // CH
        sems = (sem_a, sem_b)
        cps = [None, None]
        for c in range(nchd):
            buf = c % 2
            if cps[buf] is not None:
                cps[buf].wait()
            pltpu.sync_copy(dst3_hbm.at[wid, c], idx_v.at[buf])
            pltpu.sync_copy(x_hbm.at[pl.ds(n0 + c * CH, CH)], rows_v.at[buf])
            cps[buf] = pltpu.async_copy(
                rows_v.at[buf], xg_hbm.at[idx_v.at[buf]], sems[buf])
        pltpu.sync_copy(dstw_hbm.at[wid], idxw_v)
        pltpu.sync_copy(wf3_hbm.at[wid], wrows_v)
        cp_w = pltpu.async_copy(wrows_v, ws_hbm.at[idxw_v], sem_w)
        for cp in cps:
            if cp is not None:
                cp.wait()
        cp_w.wait()

    nch = TPW // CH2

    @functools.partial(
        pl.kernel, mesh=mesh,
        out_type=jax.ShapeDtypeStruct((N, D), jnp.float32),
        scratch_types=[pltpu.VMEM((2, CH2), jnp.int32),
                       pltpu.VMEM((2, CH2), jnp.int32),
                       pltpu.VMEM((2, CH2, D), jnp.float32),
                       pltpu.VMEM((2, CH2, D), jnp.float32),
                       pltpu.SemaphoreType.DMA,
                       pltpu.SemaphoreType.DMA],
    )
    def _combine(yg_hbm, dstT_hbm, out_hbm, i0_v, i1_v, r0_v, r1_v,
                 sem_a, sem_b):
        wid = lax.axis_index("s") * 2 + lax.axis_index("c")

        def start(c, buf):
            t0 = wid * TPW + c * CH2
            pltpu.sync_copy(dstT_hbm.at[0, pl.ds(t0, CH2)], i0_v.at[buf])
            pltpu.sync_copy(dstT_hbm.at[1, pl.ds(t0, CH2)], i1_v.at[buf])
            ca = pltpu.async_copy(yg_hbm.at[i0_v.at[buf]], r0_v.at[buf], sem_a)
            cb = pltpu.async_copy(yg_hbm.at[i1_v.at[buf]], r1_v.at[buf], sem_b)
            return ca, cb

        def finish(c, buf, cps):
            cps[0].wait()
            cps[1].wait()

            def rbody(r, carry):
                for j in range(D // 16):
                    sl = pl.ds(j * 16, 16)
                    r0_v[buf, r, sl] = r0_v[buf, r, sl] + r1_v[buf, r, sl]
                return carry

            lax.fori_loop(0, CH2, rbody, 0)
            t0 = wid * TPW + c * CH2
            pltpu.sync_copy(r0_v.at[buf], out_hbm.at[pl.ds(t0, CH2)])

        cps = start(0, 0)
        for c in range(nch):
            nxt = start(c + 1, (c + 1) % 2) if c + 1 < nch else None
            finish(c, c % 2, cps)
            cps = nxt

    return _dispatch, _combine


def kernel(x, Wg, bg, W1, b1, W2, b2):
    orig_shape = x.shape
    xf = x.reshape(-1, orig_shape[-1])
    bg2 = bg.reshape(1, E)
    w1b = W1.astype(jnp.bfloat16)
    w2b = W2.astype(jnp.bfloat16)
    b1r = b1.reshape(E, 1, DFF)
    b2r = b2.reshape(E, 1, D)

    dst, wts, bexp, bact = pl.pallas_call(
        _route_body,
        in_specs=[
            pl.BlockSpec((N, D), lambda: (0, 0)),
            pl.BlockSpec((D, E), lambda: (0, 0)),
            pl.BlockSpec((1, E), lambda: (0, 0)),
        ],
        out_specs=[
            pl.BlockSpec((N, K), lambda: (0, 0)),
            pl.BlockSpec((N, K), lambda: (0, 0)),
            pl.BlockSpec((1, NB), lambda: (0, 0)),
            pl.BlockSpec((1, NB), lambda: (0, 0)),
        ],
        out_shape=[
            jax.ShapeDtypeStruct((N, K), jnp.int32),
            jax.ShapeDtypeStruct((N, K), jnp.float32),
            jax.ShapeDtypeStruct((1, NB), jnp.int32),
            jax.ShapeDtypeStruct((1, NB), jnp.int32),
        ],
    )(xf, Wg, bg2)

    dstT = dst.T
    dst3 = dstT.reshape(NW, APW // CH, CH)
    dstw = dstT.reshape(NW, APW)
    wf3 = jnp.broadcast_to(wts.T.reshape(NW, APW, 1), (NW, APW, 128))

    _dispatch, _combine = _sc_kernels()
    xg, wsort = _dispatch(xf, dst3, dstw, wf3)

    grid_spec = pltpu.PrefetchScalarGridSpec(
        num_scalar_prefetch=2,
        grid=(NB,),
        in_specs=[
            pl.BlockSpec((BT, D), lambda b, be, ba: (b, 0)),
            pl.BlockSpec((1, D, DFF), lambda b, be, ba: (be[b], 0, 0)),
            pl.BlockSpec((1, 1, DFF), lambda b, be, ba: (be[b], 0, 0)),
            pl.BlockSpec((1, DFF, D), lambda b, be, ba: (be[b], 0, 0)),
            pl.BlockSpec((1, 1, D), lambda b, be, ba: (be[b], 0, 0)),
            pl.BlockSpec((BT, 128), lambda b, be, ba: (b, 0)),
        ],
        out_specs=pl.BlockSpec((BT, D), lambda b, be, ba: (b, 0)),
    )
    yg = pl.pallas_call(
        _ffn_body,
        grid_spec=grid_spec,
        out_shape=jax.ShapeDtypeStruct((S, D), jnp.float32),
        compiler_params=pltpu.CompilerParams(
            dimension_semantics=("arbitrary",)),
    )(bexp.reshape(NB), bact.reshape(NB), xg, w1b, b1r, w2b, b2r, wsort)

    out = _combine(yg, dstT)
    return out.reshape(orig_shape)
